# baseline (speedup 1.0000x reference)
import jax
import jax.numpy as jnp
from jax import lax
from jax.experimental import pallas as pl
from jax.experimental.pallas import tpu as pltpu

K = 20
ROWS = 1024
COLS = 100000
BLOCK_ROWS = 8
SEG = 160
NSEG = COLS // SEG
NSEL = 24
BIG = COLS


def _topk_body(x_ref, out_ref):
    x = x_ref[...]

    segmin = jnp.min(x, axis=2)

    siota = lax.broadcasted_iota(jnp.int32, (BLOCK_ROWS, NSEG), 1)

    def sel_step(j, carry):
        sm, sel = carry
        m = jnp.min(sm, axis=1, keepdims=True)
        sid = jnp.min(jnp.where(sm == m, siota, BIG), axis=1, keepdims=True)
        jcol = lax.broadcasted_iota(jnp.int32, (BLOCK_ROWS, NSEL), 1)
        sel = jnp.where(jcol == j, sid, sel)
        sm = jnp.where((sm == m) & (siota == sid), jnp.inf, sm)
        return sm, sel

    sel0 = jnp.zeros((BLOCK_ROWS, NSEL), jnp.int32)
    _, sel = lax.fori_loop(0, NSEL, sel_step, (segmin, sel0))

    onehot = (sel[:, :, None] == siota[:, None, :]).astype(jnp.float32)
    cand = lax.dot_general(onehot, x, (((2,), (1,)), ((0,), (0,))),
                           precision=lax.Precision.HIGHEST,
                           preferred_element_type=jnp.float32)
    cidx = (sel[:, :, None] * SEG
            + lax.broadcasted_iota(jnp.int32, (BLOCK_ROWS, NSEL, SEG), 2))

    kcol = lax.broadcasted_iota(jnp.int32, (BLOCK_ROWS, K), 1)

    def step(k, carry):
        cv, out = carry
        m = jnp.min(cv, axis=(1, 2), keepdims=True)
        idx = jnp.min(jnp.where(cv == m, cidx, BIG), axis=(1, 2),
                      keepdims=True)
        out = jnp.where(kcol == k, idx[:, :, 0], out)
        cv = jnp.where((cv == m) & (cidx == idx), jnp.inf, cv)
        return cv, out

    out0 = jnp.zeros((BLOCK_ROWS, K), jnp.int32)
    _, out = lax.fori_loop(0, K, step, (cand, out0))
    out_ref[...] = out


def kernel(inputs):
    x3 = inputs.reshape(ROWS, NSEG, SEG)
    return pl.pallas_call(
        _topk_body,
        grid=(ROWS // BLOCK_ROWS,),
        in_specs=[pl.BlockSpec((BLOCK_ROWS, NSEG, SEG), lambda i: (i, 0, 0))],
        out_specs=pl.BlockSpec((BLOCK_ROWS, K), lambda i: (i, 0)),
        out_shape=jax.ShapeDtypeStruct((ROWS, K), jnp.int32),
    )(x3)
